# baseline (speedup 1.0000x reference)
import jax
import jax.numpy as jnp
from jax import lax
from jax.experimental import pallas as pl
from jax.experimental.pallas import tpu as pltpu
from jax.experimental.pallas import tpu_sc as plsc

SIG_LEN = 4096
MASK_LEN = 2048
SIGMA = 0.1
N_SHIFTS = 16384

LANES = 16
NUM_CORES = 2
NUM_SUBCORES = 16
NUM_WORKERS = NUM_CORES * NUM_SUBCORES
ROWS_PER_WORKER = N_SHIFTS // NUM_WORKERS
ROW_CHUNK = 4
NUM_CHUNKS = ROWS_PER_WORKER // ROW_CHUNK
NBUF_IN = 8
NBUF_OUT = 4


def _sc_body(sig_hbm, shifts_hbm, noise_hbm, out_hbm, sig2_v, shifts_v, nbuf,
             obuf, sem_in0, sem_in1, sem_in2, sem_in3, sem_in4, sem_in5, sem_in6, sem_in7, sem_out0, sem_out1, sem_out2, sem_out3):
    wid = lax.axis_index("s") * NUM_CORES + lax.axis_index("c")
    base_row = wid * ROWS_PER_WORKER
    sems_in = (sem_in0, sem_in1, sem_in2, sem_in3, sem_in4, sem_in5, sem_in6, sem_in7)
    sems_out = (sem_out0, sem_out1, sem_out2, sem_out3)

    def noise_slice(c):
        return noise_hbm.at[pl.ds(base_row + c * ROW_CHUNK, ROW_CHUNK)]

    def out_slice(c):
        return out_hbm.at[pl.ds(base_row + c * ROW_CHUNK, ROW_CHUNK)]

    for c in range(NBUF_IN - 1):
        pltpu.async_copy(noise_slice(c), nbuf.at[c], sems_in[c])

    pltpu.sync_copy(sig_hbm, sig2_v.at[pl.ds(0, SIG_LEN)])
    pltpu.sync_copy(sig_hbm, sig2_v.at[pl.ds(SIG_LEN, SIG_LEN)])
    pltpu.sync_copy(
        shifts_hbm.at[pl.ds(base_row, ROWS_PER_WORKER)],
        shifts_v.at[pl.ds(0, ROWS_PER_WORKER)],
    )

    def outer(c4, carry):
        for bi in range(NBUF_IN):
            c = c4 * NBUF_IN + bi
            bo = bi % NBUF_OUT

            @pl.when(c + NBUF_IN - 1 < NUM_CHUNKS)
            def _start_next():
                nxt = (bi + NBUF_IN - 1) % NBUF_IN
                pltpu.async_copy(noise_slice(c + NBUF_IN - 1), nbuf.at[nxt],
                                 sems_in[nxt])

            pltpu.make_async_copy(noise_slice(c), nbuf.at[bi], sems_in[bi]).wait()

            @pl.when(c >= NBUF_OUT)
            def _wait_out():
                pltpu.make_async_copy(obuf.at[bo], out_slice(c), sems_out[bo]).wait()

            sv = shifts_v[pl.ds(c * ROW_CHUNK, LANES)]
            for r in range(ROW_CHUNK):
                shift = sv[r]

                @plsc.parallel_loop(0, MASK_LEN, LANES, unroll=8)
                def j_body(off, bi=bi, bo=bo, r=r, shift=shift):
                    w = sig2_v[pl.ds(shift + off, LANES)]
                    n = nbuf[bi, r, pl.ds(off, LANES)]
                    obuf[bo, r, pl.ds(off, LANES)] = w + SIGMA * n
            pltpu.async_copy(obuf.at[bo], out_slice(c), sems_out[bo])
        return carry

    lax.fori_loop(0, NUM_CHUNKS // NBUF_IN, outer, 0)
    for b in range(NBUF_OUT):
        pltpu.make_async_copy(
            obuf.at[b], out_slice(NUM_CHUNKS - NBUF_OUT + b), sems_out[b]
        ).wait()


@jax.jit
def kernel(sig, shifts, noise):
    mesh = plsc.VectorSubcoreMesh(
        core_axis_name="c", subcore_axis_name="s",
        num_cores=NUM_CORES, num_subcores=NUM_SUBCORES,
    )
    run = pl.kernel(
        _sc_body,
        out_type=jax.ShapeDtypeStruct((N_SHIFTS, MASK_LEN), jnp.float32),
        mesh=mesh,
        scratch_types=[
            pltpu.VMEM((2 * SIG_LEN,), jnp.float32),
            pltpu.VMEM((ROWS_PER_WORKER + LANES,), jnp.int32),
            pltpu.VMEM((NBUF_IN, ROW_CHUNK, MASK_LEN), jnp.float32),
            pltpu.VMEM((NBUF_OUT, ROW_CHUNK, MASK_LEN), jnp.float32),
            pltpu.SemaphoreType.DMA,
            pltpu.SemaphoreType.DMA,
            pltpu.SemaphoreType.DMA,
            pltpu.SemaphoreType.DMA,
            pltpu.SemaphoreType.DMA,
            pltpu.SemaphoreType.DMA,
            pltpu.SemaphoreType.DMA,
            pltpu.SemaphoreType.DMA,
            pltpu.SemaphoreType.DMA,
            pltpu.SemaphoreType.DMA,
            pltpu.SemaphoreType.DMA,
            pltpu.SemaphoreType.DMA,
        ],
    )
    return run(sig, shifts.astype(jnp.int32), noise)


# speedup vs baseline: 1.0107x; 1.0107x over previous
import jax
import jax.numpy as jnp
from jax import lax
from jax.experimental import pallas as pl
from jax.experimental.pallas import tpu as pltpu
from jax.experimental.pallas import tpu_sc as plsc

SIG_LEN = 4096
MASK_LEN = 2048
SIGMA = 0.1
N_SHIFTS = 16384

LANES = 16
NUM_CORES = 2
NUM_SUBCORES = 16
NUM_WORKERS = NUM_CORES * NUM_SUBCORES
ROWS_PER_WORKER = N_SHIFTS // NUM_WORKERS
ROW_CHUNK = 4
NUM_CHUNKS = ROWS_PER_WORKER // ROW_CHUNK
NBUF_IN = 4
NBUF_OUT = 4


def _sc_body(sig_hbm, shifts_hbm, noise_hbm, out_hbm, sig2_v, shifts_v, nbuf,
             obuf, sem_in0, sem_in1, sem_in2, sem_in3, sem_out0, sem_out1, sem_out2, sem_out3):
    wid = lax.axis_index("s") * NUM_CORES + lax.axis_index("c")
    base_row = wid * ROWS_PER_WORKER
    sems_in = (sem_in0, sem_in1, sem_in2, sem_in3)
    sems_out = (sem_out0, sem_out1, sem_out2, sem_out3)

    def noise_slice(c):
        return noise_hbm.at[pl.ds(base_row + c * ROW_CHUNK, ROW_CHUNK)]

    def out_slice(c):
        return out_hbm.at[pl.ds(base_row + c * ROW_CHUNK, ROW_CHUNK)]

    for c in range(NBUF_IN - 1):
        pltpu.async_copy(noise_slice(c), nbuf.at[c], sems_in[c])

    pltpu.sync_copy(sig_hbm, sig2_v.at[pl.ds(0, SIG_LEN)])
    pltpu.sync_copy(sig_hbm, sig2_v.at[pl.ds(SIG_LEN, SIG_LEN)])
    pltpu.sync_copy(
        shifts_hbm.at[pl.ds(base_row, ROWS_PER_WORKER)],
        shifts_v.at[pl.ds(0, ROWS_PER_WORKER)],
    )

    def outer(c4, carry):
        for bi in range(NBUF_IN):
            c = c4 * NBUF_IN + bi
            bo = bi % NBUF_OUT

            @pl.when(c + NBUF_IN - 1 < NUM_CHUNKS)
            def _start_next():
                nxt = (bi + NBUF_IN - 1) % NBUF_IN
                pltpu.async_copy(noise_slice(c + NBUF_IN - 1), nbuf.at[nxt],
                                 sems_in[nxt])

            pltpu.make_async_copy(noise_slice(c), nbuf.at[bi], sems_in[bi]).wait()

            @pl.when(c >= NBUF_OUT)
            def _wait_out():
                pltpu.make_async_copy(obuf.at[bo], out_slice(c), sems_out[bo]).wait()

            sv = shifts_v[pl.ds(c * ROW_CHUNK, LANES)]
            for r in range(ROW_CHUNK):
                shift = sv[r]

                @plsc.parallel_loop(0, MASK_LEN, LANES, unroll=8)
                def j_body(off, bi=bi, bo=bo, r=r, shift=shift):
                    w = sig2_v[pl.ds(shift + off, LANES)]
                    n = nbuf[bi, r, pl.ds(off, LANES)]
                    obuf[bo, r, pl.ds(off, LANES)] = w + SIGMA * n
            pltpu.async_copy(obuf.at[bo], out_slice(c), sems_out[bo])
        return carry

    lax.fori_loop(0, NUM_CHUNKS // NBUF_IN, outer, 0)
    for b in range(NBUF_OUT):
        pltpu.make_async_copy(
            obuf.at[b], out_slice(NUM_CHUNKS - NBUF_OUT + b), sems_out[b]
        ).wait()


@jax.jit
def kernel(sig, shifts, noise):
    mesh = plsc.VectorSubcoreMesh(
        core_axis_name="c", subcore_axis_name="s",
        num_cores=NUM_CORES, num_subcores=NUM_SUBCORES,
    )
    run = pl.kernel(
        _sc_body,
        out_type=jax.ShapeDtypeStruct((N_SHIFTS, MASK_LEN), jnp.float32),
        mesh=mesh,
        scratch_types=[
            pltpu.VMEM((2 * SIG_LEN,), jnp.float32),
            pltpu.VMEM((ROWS_PER_WORKER + LANES,), jnp.int32),
            pltpu.VMEM((NBUF_IN, ROW_CHUNK, MASK_LEN), jnp.float32),
            pltpu.VMEM((NBUF_OUT, ROW_CHUNK, MASK_LEN), jnp.float32),
            pltpu.SemaphoreType.DMA,
            pltpu.SemaphoreType.DMA,
            pltpu.SemaphoreType.DMA,
            pltpu.SemaphoreType.DMA,
            pltpu.SemaphoreType.DMA,
            pltpu.SemaphoreType.DMA,
            pltpu.SemaphoreType.DMA,
            pltpu.SemaphoreType.DMA,
        ],
    )
    return run(sig, shifts.astype(jnp.int32), noise)
